# baseline (speedup 1.0000x reference)
import jax
import jax.numpy as jnp
from jax import lax
from jax.experimental import pallas as pl
from jax.experimental.pallas import tpu as pltpu
from jax.experimental.pallas import tpu_sc as plsc

B, S, D = 4, 2048, 1024
NUM_STATES = 64
ROWS = B * S
R_BLK = 2048
NC = 2


def _gather_kernel(idx_hbm, cv_hbm, o_hbm, idx_v, adj_v, gsem):
    wid = lax.axis_index("s") * NC + lax.axis_index("c")

    @pl.when(wid == 0)
    def _():
        pltpu.sync_copy(idx_hbm, idx_v)
        pltpu.async_copy(cv_hbm.at[idx_v.at[pl.ds(0, B)]], adj_v, gsem).wait()
        pltpu.sync_copy(adj_v, o_hbm)


def _sc_gather(idx_pad, cv):
    mesh = plsc.VectorSubcoreMesh(core_axis_name="c", subcore_axis_name="s")
    run = pl.kernel(
        _gather_kernel,
        out_type=jax.ShapeDtypeStruct((B, D), jnp.float32),
        mesh=mesh,
        scratch_types=[
            pltpu.VMEM((8,), jnp.int32),
            pltpu.VMEM((B, D), jnp.float32),
            pltpu.SemaphoreType.DMA,
        ],
    )
    return run(idx_pad, cv)


def _add_kernel(h_ref, adj_ref, o_ref):
    o_ref[...] = h_ref[...] + adj_ref[0]


def kernel(hidden_states, affective_state_indices, control_vectors):
    idx_pad = jnp.zeros((8,), jnp.int32).at[:B].set(
        affective_state_indices.astype(jnp.int32))
    adj = _sc_gather(idx_pad, control_vectors)
    adj3 = adj.reshape(B, 1, D)

    h2d = hidden_states.reshape(ROWS, D)
    blks_per_batch = S // R_BLK
    out = pl.pallas_call(
        _add_kernel,
        grid=(ROWS // R_BLK,),
        in_specs=[
            pl.BlockSpec((R_BLK, D), lambda i: (i, 0)),
            pl.BlockSpec((1, 1, D), lambda i: (i // blks_per_batch, 0, 0)),
        ],
        out_specs=pl.BlockSpec((R_BLK, D), lambda i: (i, 0)),
        out_shape=jax.ShapeDtypeStruct((ROWS, D), jnp.float32),
    )(h2d, adj3)
    return out.reshape(B, S, D)


# speedup vs baseline: 1.8228x; 1.8228x over previous
import jax
import jax.numpy as jnp
from jax.experimental import pallas as pl
from jax.experimental.pallas import tpu as pltpu

B, S, D = 4, 2048, 1024
NUM_STATES = 64
ROWS = B * S
R_BLK = 2048


def _add_kernel(idx_ref, h_ref, cv_ref, o_ref):
    o_ref[...] = h_ref[...] + cv_ref[0]


def kernel(hidden_states, affective_state_indices, control_vectors):
    idx = affective_state_indices.astype(jnp.int32)
    h2d = hidden_states.reshape(ROWS, D)
    cv3 = control_vectors.reshape(NUM_STATES, 1, D)
    blks_per_batch = S // R_BLK
    out = pl.pallas_call(
        _add_kernel,
        grid_spec=pltpu.PrefetchScalarGridSpec(
            num_scalar_prefetch=1,
            grid=(ROWS // R_BLK,),
            in_specs=[
                pl.BlockSpec((R_BLK, D), lambda i, idx_ref: (i, 0)),
                pl.BlockSpec(
                    (1, 1, D),
                    lambda i, idx_ref: (idx_ref[i // blks_per_batch], 0, 0)),
            ],
            out_specs=pl.BlockSpec((R_BLK, D), lambda i, idx_ref: (i, 0)),
        ),
        out_shape=jax.ShapeDtypeStruct((ROWS, D), jnp.float32),
    )(idx, h2d, cv3)
    return out.reshape(B, S, D)
